# baseline (speedup 1.0000x reference)
import functools

import jax
import jax.numpy as jnp
import numpy as np
from jax.experimental import pallas as pl

_TINY = np.float32(np.finfo(np.float32).tiny)
_KS0 = np.uint32(0)
_KS1 = np.uint32(42)
_KS2 = np.uint32(0x1BD11BDA) ^ _KS1
_NEG_INF = np.float32(-np.inf)


def _rotl(x, d):
    return (x << np.uint32(d)) | (x >> np.uint32(32 - d))


def _threefry_bits(x1_init):
    x1 = x1_init
    x0 = x1
    t = _rotl(x1, 13)
    x1 = x0 ^ t

    def rounds(x0, x1, rots):
        for r in rots:
            x0 = x0 + x1
            x1 = _rotl(x1, r)
            x1 = x0 ^ x1
        return x0, x1

    x0, x1 = rounds(x0, x1, (15, 26, 6))
    x0, x1 = x0 + _KS1, x1 + (_KS2 + np.uint32(1))
    x0, x1 = rounds(x0, x1, (17, 29, 16, 24))
    x0, x1 = x0 + _KS2, x1 + (_KS0 + np.uint32(2))
    x0, x1 = rounds(x0, x1, (13, 15, 26, 6))
    x0, x1 = x0 + _KS0, x1 + (_KS1 + np.uint32(3))
    x0, x1 = rounds(x0, x1, (17, 29, 16, 24))
    x0, x1 = x0 + _KS1, x1 + (_KS2 + np.uint32(4))
    x0, x1 = rounds(x0, x1, (13, 15, 26, 6))
    x0, x1 = x0 + _KS2, x1 + (_KS0 + np.uint32(5))
    return x0 ^ x1


def _gumbel_plus(x, x1_init):
    bits = _threefry_bits(x1_init)
    float_bits = (bits >> np.uint32(9)) | np.uint32(0x3F800000)
    u0 = jax.lax.bitcast_convert_type(float_bits, jnp.float32) - np.float32(1.0)
    u = jnp.maximum(_TINY, u0 * (np.float32(1.0) - _TINY) + _TINY)
    g = -jnp.log(-jnp.log(u))
    return g + x


def _group_kernel(x_ref, o_ref, *, rows_per_grp, sub, chunk, vocab, lanes):
    grp = pl.program_id(0)
    t = rows_per_grp * sub
    nfull = chunk // lanes
    tail = chunk - nfull * lanes

    qi = jax.lax.broadcasted_iota(jnp.uint32, (t, lanes), 0)
    li = jax.lax.broadcasted_iota(jnp.uint32, (t, lanes), 1)
    q = qi // np.uint32(sub)
    s = qi % np.uint32(sub)
    pat = q * np.uint32(vocab) + s * np.uint32(chunk) + li + _KS1
    vpat = (s * np.uint32(chunk) + li).astype(jnp.int32)

    base = jnp.uint32(grp) * np.uint32(rows_per_grp * vocab)
    big = np.int32(vocab)

    def body(c, carry):
        run_max, run_idx = carry
        off = c * lanes
        x = x_ref[0, :, pl.ds(off, lanes)]
        y = _gumbel_plus(x, pat + (base + off.astype(jnp.uint32)))
        v = vpat + off.astype(jnp.int32)
        upd = y > run_max
        run_idx = jnp.where(upd, v, run_idx)
        run_max = jnp.maximum(run_max, y)
        return run_max, run_idx

    init = (jnp.full((t, lanes), _NEG_INF, jnp.float32),
            jnp.full((t, lanes), big, jnp.int32))
    run_max, run_idx = jax.lax.fori_loop(0, nfull, body, init)

    off_t = nfull * lanes
    x_t = x_ref[0, :, pl.ds(off_t, tail)]
    y_t = _gumbel_plus(x_t, pat[:, :tail] + (base + np.uint32(off_t)))
    v_t = vpat[:, :tail] + np.int32(off_t)
    pad = lanes - tail
    y_full = jnp.concatenate(
        [y_t, jnp.full((t, pad), _NEG_INF, jnp.float32)], axis=1)
    v_full = jnp.concatenate([v_t, jnp.full((t, pad), big, jnp.int32)], axis=1)
    upd_t = y_full > run_max
    run_idx = jnp.where(upd_t, v_full, run_idx)
    run_max = jnp.maximum(run_max, y_full)

    acts = []
    for r in range(rows_per_grp):
        rm = run_max[r * sub:(r + 1) * sub]
        ri = run_idx[r * sub:(r + 1) * sub]
        m = jnp.max(rm)
        acts.append(jnp.min(jnp.where(rm == m, ri, big)))

    a32 = jnp.full((t, 1), acts[0], jnp.int32)
    qcol = jax.lax.broadcasted_iota(jnp.int32, (t, 1), 0) // np.int32(sub)
    for r in range(1, rows_per_grp):
        a32 = jnp.where(qcol == r, acts[r], a32)

    one = np.float32(1.0)
    zero = np.float32(0.0)

    def obody(c, _):
        off = c * lanes
        v = vpat + off.astype(jnp.int32)
        o_ref[0, :, pl.ds(off, lanes)] = jnp.where(v == a32, one, zero)
        return 0

    jax.lax.fori_loop(0, nfull, obody, 0)
    v_tl = vpat[:, :tail] + np.int32(off_t)
    o_ref[0, :, pl.ds(off_t, tail)] = jnp.where(v_tl == a32, one, zero)


def kernel(x):
    m, n, vocab = x.shape
    rows = m * n
    sub = 8
    rows_per_grp = 4
    grps = rows // rows_per_grp
    chunk = vocab // sub
    t = rows_per_grp * sub
    xr = x.reshape(grps, t, chunk)
    out = pl.pallas_call(
        functools.partial(_group_kernel, rows_per_grp=rows_per_grp, sub=sub,
                          chunk=chunk, vocab=vocab, lanes=128),
        grid=(grps,),
        in_specs=[pl.BlockSpec((1, t, chunk), lambda g: (g, 0, 0))],
        out_specs=pl.BlockSpec((1, t, chunk), lambda g: (g, 0, 0)),
        out_shape=jax.ShapeDtypeStruct((grps, t, chunk), jnp.float32),
    )(xr)
    return out.reshape(m, n, vocab)


# speedup vs baseline: 1.4330x; 1.4330x over previous
import functools

import jax
import jax.numpy as jnp
import numpy as np
from jax.experimental import pallas as pl

_TINY = np.float32(np.finfo(np.float32).tiny)
_KS0 = np.uint32(0)
_KS1 = np.uint32(42)
_KS2 = np.uint32(0x1BD11BDA) ^ _KS1
_NEG_INF = np.float32(-np.inf)


def _rotl(x, d):
    return (x << np.uint32(d)) | (x >> np.uint32(32 - d))


def _threefry_bits(x1_init):
    x1 = x1_init
    x0 = x1
    t = _rotl(x1, 13)
    x1 = x0 ^ t

    def rounds(x0, x1, rots):
        for r in rots:
            x0 = x0 + x1
            x1 = _rotl(x1, r)
            x1 = x0 ^ x1
        return x0, x1

    x0, x1 = rounds(x0, x1, (15, 26, 6))
    x0, x1 = x0 + _KS1, x1 + (_KS2 + np.uint32(1))
    x0, x1 = rounds(x0, x1, (17, 29, 16, 24))
    x0, x1 = x0 + _KS2, x1 + (_KS0 + np.uint32(2))
    x0, x1 = rounds(x0, x1, (13, 15, 26, 6))
    x0, x1 = x0 + _KS0, x1 + (_KS1 + np.uint32(3))
    x0, x1 = rounds(x0, x1, (17, 29, 16, 24))
    x0, x1 = x0 + _KS1, x1 + (_KS2 + np.uint32(4))
    x0, x1 = rounds(x0, x1, (13, 15, 26, 6))
    x0, x1 = x0 + _KS2, x1 + (_KS0 + np.uint32(5))
    return x0 ^ x1


def _gumbel_plus(x, x1_init):
    bits = _threefry_bits(x1_init)
    float_bits = (bits >> np.uint32(9)) | np.uint32(0x3F800000)
    u0 = jax.lax.bitcast_convert_type(float_bits, jnp.float32) - np.float32(1.0)
    u = jnp.maximum(_TINY, u0 * (np.float32(1.0) - _TINY) + _TINY)
    g = -jnp.log(-jnp.log(u))
    return g + x


def _group_kernel(x_ref, o_ref, *, rows_per_grp, sub, chunk, vocab, lanes):
    grp = pl.program_id(0)
    t = rows_per_grp * sub
    nfull = chunk // lanes
    tail = chunk - nfull * lanes

    qi = jax.lax.broadcasted_iota(jnp.uint32, (t, lanes), 0)
    li = jax.lax.broadcasted_iota(jnp.uint32, (t, lanes), 1)
    q = qi // np.uint32(sub)
    s = qi % np.uint32(sub)
    pat = q * np.uint32(vocab) + s * np.uint32(chunk) + li + _KS1
    vpat = (s * np.uint32(chunk) + li).astype(jnp.int32)

    base = jnp.uint32(grp) * np.uint32(rows_per_grp * vocab)
    big = np.int32(vocab)

    def body(c, carry):
        run_max, run_idx = carry
        off = c * lanes
        x = x_ref[0, :, pl.ds(off, lanes)]
        y = _gumbel_plus(x, pat + (base + off.astype(jnp.uint32)))
        v = vpat + off.astype(jnp.int32)
        upd = y > run_max
        run_idx = jnp.where(upd, v, run_idx)
        run_max = jnp.maximum(run_max, y)
        return run_max, run_idx

    init = (jnp.full((t, lanes), _NEG_INF, jnp.float32),
            jnp.full((t, lanes), big, jnp.int32))
    run_max, run_idx = jax.lax.fori_loop(0, nfull, body, init)

    off_t = nfull * lanes
    x_t = x_ref[0, :, pl.ds(off_t, tail)]
    y_t = _gumbel_plus(x_t, pat[:, :tail] + (base + np.uint32(off_t)))
    v_t = vpat[:, :tail] + np.int32(off_t)
    pad = lanes - tail
    y_full = jnp.concatenate(
        [y_t, jnp.full((t, pad), _NEG_INF, jnp.float32)], axis=1)
    v_full = jnp.concatenate([v_t, jnp.full((t, pad), big, jnp.int32)], axis=1)
    upd_t = y_full > run_max
    run_idx = jnp.where(upd_t, v_full, run_idx)
    run_max = jnp.maximum(run_max, y_full)

    acts = []
    for r in range(rows_per_grp):
        rm = run_max[r * sub:(r + 1) * sub]
        ri = run_idx[r * sub:(r + 1) * sub]
        m = jnp.max(rm)
        acts.append(jnp.min(jnp.where(rm == m, ri, big)))

    a32 = jnp.full((t, 1), acts[0], jnp.int32)
    qcol = jax.lax.broadcasted_iota(jnp.int32, (t, 1), 0) // np.int32(sub)
    for r in range(1, rows_per_grp):
        a32 = jnp.where(qcol == r, acts[r], a32)

    one = np.float32(1.0)
    zero = np.float32(0.0)

    def obody(c, _):
        off = c * lanes
        v = vpat + off.astype(jnp.int32)
        o_ref[0, :, pl.ds(off, lanes)] = jnp.where(v == a32, one, zero)
        return 0

    jax.lax.fori_loop(0, nfull, obody, 0)
    v_tl = vpat[:, :tail] + np.int32(off_t)
    o_ref[0, :, pl.ds(off_t, tail)] = jnp.where(v_tl == a32, one, zero)


def kernel(x):
    m, n, vocab = x.shape
    rows = m * n
    sub = 8
    rows_per_grp = 4
    grps = rows // rows_per_grp
    chunk = vocab // sub
    t = rows_per_grp * sub
    xr = x.reshape(grps, t, chunk)
    out = pl.pallas_call(
        functools.partial(_group_kernel, rows_per_grp=rows_per_grp, sub=sub,
                          chunk=chunk, vocab=vocab, lanes=512),
        grid=(grps,),
        in_specs=[pl.BlockSpec((1, t, chunk), lambda g: (g, 0, 0))],
        out_specs=pl.BlockSpec((1, t, chunk), lambda g: (g, 0, 0)),
        out_shape=jax.ShapeDtypeStruct((grps, t, chunk), jnp.float32),
    )(xr)
    return out.reshape(m, n, vocab)


# speedup vs baseline: 1.5453x; 1.0784x over previous
import functools

import jax
import jax.numpy as jnp
import numpy as np
from jax.experimental import pallas as pl

_TINY = np.float32(np.finfo(np.float32).tiny)
_KS0 = np.uint32(0)
_KS1 = np.uint32(42)
_KS2 = np.uint32(0x1BD11BDA) ^ _KS1
_NEG_INF = np.float32(-np.inf)


def _rotl(x, d):
    return (x << np.uint32(d)) | (x >> np.uint32(32 - d))


def _threefry_bits(x1_init):
    x1 = x1_init
    x0 = x1
    t = _rotl(x1, 13)
    x1 = x0 ^ t

    def rounds(x0, x1, rots):
        for r in rots:
            x0 = x0 + x1
            x1 = _rotl(x1, r)
            x1 = x0 ^ x1
        return x0, x1

    x0, x1 = rounds(x0, x1, (15, 26, 6))
    x0, x1 = x0 + _KS1, x1 + (_KS2 + np.uint32(1))
    x0, x1 = rounds(x0, x1, (17, 29, 16, 24))
    x0, x1 = x0 + _KS2, x1 + (_KS0 + np.uint32(2))
    x0, x1 = rounds(x0, x1, (13, 15, 26, 6))
    x0, x1 = x0 + _KS0, x1 + (_KS1 + np.uint32(3))
    x0, x1 = rounds(x0, x1, (17, 29, 16, 24))
    x0, x1 = x0 + _KS1, x1 + (_KS2 + np.uint32(4))
    x0, x1 = rounds(x0, x1, (13, 15, 26, 6))
    x0, x1 = x0 + _KS2, x1 + (_KS0 + np.uint32(5))
    return x0 ^ x1


def _gumbel_plus(x, x1_init):
    bits = _threefry_bits(x1_init)
    float_bits = (bits >> np.uint32(9)) | np.uint32(0x3F800000)
    u0 = jax.lax.bitcast_convert_type(float_bits, jnp.float32) - np.float32(1.0)
    u = jnp.maximum(_TINY, u0 * (np.float32(1.0) - _TINY) + _TINY)
    g = -jnp.log(-jnp.log(u))
    return g + x


def _group_kernel(x_ref, o_ref, *, rows_per_grp, sub, chunk, vocab, lanes):
    grp = pl.program_id(0)
    t = rows_per_grp * sub
    nfull = chunk // lanes
    tail = chunk - nfull * lanes

    qi = jax.lax.broadcasted_iota(jnp.uint32, (t, lanes), 0)
    li = jax.lax.broadcasted_iota(jnp.uint32, (t, lanes), 1)
    q = qi // np.uint32(sub)
    s = qi % np.uint32(sub)
    pat = q * np.uint32(vocab) + s * np.uint32(chunk) + li + _KS1
    vpat = (s * np.uint32(chunk) + li).astype(jnp.int32)

    base = jnp.uint32(grp) * np.uint32(rows_per_grp * vocab)
    big = np.int32(vocab)

    run_max = jnp.full((t, lanes), _NEG_INF, jnp.float32)
    run_idx = jnp.full((t, lanes), big, jnp.int32)
    for c in range(nfull):
        off = c * lanes
        x = x_ref[0, :, off:off + lanes]
        y = _gumbel_plus(x, pat + (base + np.uint32(off)))
        v = vpat + np.int32(off)
        upd = y > run_max
        run_idx = jnp.where(upd, v, run_idx)
        run_max = jnp.maximum(run_max, y)

    off_t = nfull * lanes
    x_t = x_ref[0, :, pl.ds(off_t, tail)]
    y_t = _gumbel_plus(x_t, pat[:, :tail] + (base + np.uint32(off_t)))
    v_t = vpat[:, :tail] + np.int32(off_t)
    pad = lanes - tail
    y_full = jnp.concatenate(
        [y_t, jnp.full((t, pad), _NEG_INF, jnp.float32)], axis=1)
    v_full = jnp.concatenate([v_t, jnp.full((t, pad), big, jnp.int32)], axis=1)
    upd_t = y_full > run_max
    run_idx = jnp.where(upd_t, v_full, run_idx)
    run_max = jnp.maximum(run_max, y_full)

    acts = []
    for r in range(rows_per_grp):
        rm = run_max[r * sub:(r + 1) * sub]
        ri = run_idx[r * sub:(r + 1) * sub]
        m = jnp.max(rm)
        acts.append(jnp.min(jnp.where(rm == m, ri, big)))

    a32 = jnp.full((t, 1), acts[0], jnp.int32)
    qcol = jax.lax.broadcasted_iota(jnp.int32, (t, 1), 0) // np.int32(sub)
    for r in range(1, rows_per_grp):
        a32 = jnp.where(qcol == r, acts[r], a32)

    one = np.float32(1.0)
    zero = np.float32(0.0)

    for c in range(nfull):
        off = c * lanes
        v = vpat + np.int32(off)
        o_ref[0, :, off:off + lanes] = jnp.where(v == a32, one, zero)
    v_tl = vpat[:, :tail] + np.int32(off_t)
    o_ref[0, :, pl.ds(off_t, tail)] = jnp.where(v_tl == a32, one, zero)


def kernel(x):
    m, n, vocab = x.shape
    rows = m * n
    sub = 8
    rows_per_grp = 2
    grps = rows // rows_per_grp
    chunk = vocab // sub
    t = rows_per_grp * sub
    xr = x.reshape(grps, t, chunk)
    out = pl.pallas_call(
        functools.partial(_group_kernel, rows_per_grp=rows_per_grp, sub=sub,
                          chunk=chunk, vocab=vocab, lanes=512),
        grid=(grps,),
        in_specs=[pl.BlockSpec((1, t, chunk), lambda g: (g, 0, 0))],
        out_specs=pl.BlockSpec((1, t, chunk), lambda g: (g, 0, 0)),
        out_shape=jax.ShapeDtypeStruct((grps, t, chunk), jnp.float32),
    )(xr)
    return out.reshape(m, n, vocab)


# speedup vs baseline: 1.5459x; 1.0004x over previous
import functools

import jax
import jax.numpy as jnp
import numpy as np
from jax.experimental import pallas as pl
from jax.experimental.pallas import tpu as pltpu

_TINY = np.float32(np.finfo(np.float32).tiny)
_KS0 = np.uint32(0)
_KS1 = np.uint32(42)
_KS2 = np.uint32(0x1BD11BDA) ^ _KS1
_NEG_INF = np.float32(-np.inf)


def _rotl(x, d):
    return (x << np.uint32(d)) | (x >> np.uint32(32 - d))


def _threefry_bits(x1_init):
    x1 = x1_init
    x0 = x1
    t = _rotl(x1, 13)
    x1 = x0 ^ t

    def rounds(x0, x1, rots):
        for r in rots:
            x0 = x0 + x1
            x1 = _rotl(x1, r)
            x1 = x0 ^ x1
        return x0, x1

    x0, x1 = rounds(x0, x1, (15, 26, 6))
    x0, x1 = x0 + _KS1, x1 + (_KS2 + np.uint32(1))
    x0, x1 = rounds(x0, x1, (17, 29, 16, 24))
    x0, x1 = x0 + _KS2, x1 + (_KS0 + np.uint32(2))
    x0, x1 = rounds(x0, x1, (13, 15, 26, 6))
    x0, x1 = x0 + _KS0, x1 + (_KS1 + np.uint32(3))
    x0, x1 = rounds(x0, x1, (17, 29, 16, 24))
    x0, x1 = x0 + _KS1, x1 + (_KS2 + np.uint32(4))
    x0, x1 = rounds(x0, x1, (13, 15, 26, 6))
    x0, x1 = x0 + _KS2, x1 + (_KS0 + np.uint32(5))
    return x0 ^ x1


def _gumbel_plus(x, x1_init):
    bits = _threefry_bits(x1_init)
    float_bits = (bits >> np.uint32(9)) | np.uint32(0x3F800000)
    u0 = jax.lax.bitcast_convert_type(float_bits, jnp.float32) - np.float32(1.0)
    u = jnp.maximum(_TINY, u0 * (np.float32(1.0) - _TINY) + _TINY)
    g = -jnp.log(-jnp.log(u))
    return g + x


def _group_kernel(x_ref, o_ref, *, rows_per_grp, sub, chunk, vocab, lanes):
    grp = pl.program_id(0)
    t = rows_per_grp * sub
    nfull = chunk // lanes
    tail = chunk - nfull * lanes

    qi = jax.lax.broadcasted_iota(jnp.uint32, (t, lanes), 0)
    li = jax.lax.broadcasted_iota(jnp.uint32, (t, lanes), 1)
    q = qi // np.uint32(sub)
    s = qi % np.uint32(sub)
    pat = q * np.uint32(vocab) + s * np.uint32(chunk) + li + _KS1
    vpat = (s * np.uint32(chunk) + li).astype(jnp.int32)

    base = jnp.uint32(grp) * np.uint32(rows_per_grp * vocab)
    big = np.int32(vocab)

    run_max = jnp.full((t, lanes), _NEG_INF, jnp.float32)
    run_idx = jnp.full((t, lanes), big, jnp.int32)
    for c in range(nfull):
        off = c * lanes
        x = x_ref[0, :, off:off + lanes]
        y = _gumbel_plus(x, pat + (base + np.uint32(off)))
        v = vpat + np.int32(off)
        upd = y > run_max
        run_idx = jnp.where(upd, v, run_idx)
        run_max = jnp.maximum(run_max, y)

    off_t = nfull * lanes
    x_t = x_ref[0, :, pl.ds(off_t, tail)]
    y_t = _gumbel_plus(x_t, pat[:, :tail] + (base + np.uint32(off_t)))
    v_t = vpat[:, :tail] + np.int32(off_t)
    pad = lanes - tail
    y_full = jnp.concatenate(
        [y_t, jnp.full((t, pad), _NEG_INF, jnp.float32)], axis=1)
    v_full = jnp.concatenate([v_t, jnp.full((t, pad), big, jnp.int32)], axis=1)
    upd_t = y_full > run_max
    run_idx = jnp.where(upd_t, v_full, run_idx)
    run_max = jnp.maximum(run_max, y_full)

    acts = []
    for r in range(rows_per_grp):
        rm = run_max[r * sub:(r + 1) * sub]
        ri = run_idx[r * sub:(r + 1) * sub]
        m = jnp.max(rm)
        acts.append(jnp.min(jnp.where(rm == m, ri, big)))

    a32 = jnp.full((t, 1), acts[0], jnp.int32)
    qcol = jax.lax.broadcasted_iota(jnp.int32, (t, 1), 0) // np.int32(sub)
    for r in range(1, rows_per_grp):
        a32 = jnp.where(qcol == r, acts[r], a32)

    one = np.float32(1.0)
    zero = np.float32(0.0)

    for c in range(nfull):
        off = c * lanes
        v = vpat + np.int32(off)
        o_ref[0, :, off:off + lanes] = jnp.where(v == a32, one, zero)
    v_tl = vpat[:, :tail] + np.int32(off_t)
    o_ref[0, :, pl.ds(off_t, tail)] = jnp.where(v_tl == a32, one, zero)


def kernel(x):
    m, n, vocab = x.shape
    rows = m * n
    sub = 8
    rows_per_grp = 2
    grps = rows // rows_per_grp
    chunk = vocab // sub
    t = rows_per_grp * sub
    xr = x.reshape(grps, t, chunk)
    out = pl.pallas_call(
        functools.partial(_group_kernel, rows_per_grp=rows_per_grp, sub=sub,
                          chunk=chunk, vocab=vocab, lanes=512),
        grid=(grps,),
        in_specs=[pl.BlockSpec((1, t, chunk), lambda g: (g, 0, 0))],
        out_specs=pl.BlockSpec((1, t, chunk), lambda g: (g, 0, 0)),
        out_shape=jax.ShapeDtypeStruct((grps, t, chunk), jnp.float32),
        compiler_params=pltpu.CompilerParams(
            dimension_semantics=("parallel",)),
    )(xr)
    return out.reshape(m, n, vocab)


# speedup vs baseline: 1.5523x; 1.0041x over previous
import functools

import jax
import jax.numpy as jnp
import numpy as np
from jax.experimental import pallas as pl
from jax.experimental.pallas import tpu as pltpu

_TINY = np.float32(np.finfo(np.float32).tiny)
_KS0 = np.uint32(0)
_KS1 = np.uint32(42)
_KS2 = np.uint32(0x1BD11BDA) ^ _KS1
_NEG_INF = np.float32(-np.inf)


def _rotl(x, d):
    return (x << np.uint32(d)) | (x >> np.uint32(32 - d))


def _threefry_bits(x1_init):
    x1 = x1_init
    x0 = x1
    t = _rotl(x1, 13)
    x1 = x0 ^ t

    def rounds(x0, x1, rots):
        for r in rots:
            x0 = x0 + x1
            x1 = _rotl(x1, r)
            x1 = x0 ^ x1
        return x0, x1

    x0, x1 = rounds(x0, x1, (15, 26, 6))
    x0, x1 = x0 + _KS1, x1 + (_KS2 + np.uint32(1))
    x0, x1 = rounds(x0, x1, (17, 29, 16, 24))
    x0, x1 = x0 + _KS2, x1 + (_KS0 + np.uint32(2))
    x0, x1 = rounds(x0, x1, (13, 15, 26, 6))
    x0, x1 = x0 + _KS0, x1 + (_KS1 + np.uint32(3))
    x0, x1 = rounds(x0, x1, (17, 29, 16, 24))
    x0, x1 = x0 + _KS1, x1 + (_KS2 + np.uint32(4))
    x0, x1 = rounds(x0, x1, (13, 15, 26, 6))
    x0, x1 = x0 + _KS2, x1 + (_KS0 + np.uint32(5))
    return x0 ^ x1


def _gumbel_plus(x, x1_init):
    bits = _threefry_bits(x1_init)
    float_bits = (bits >> np.uint32(9)) | np.uint32(0x3F800000)
    u0 = jax.lax.bitcast_convert_type(float_bits, jnp.float32) - np.float32(1.0)
    u = jnp.maximum(_TINY, u0 * (np.float32(1.0) - _TINY) + _TINY)
    g = -jnp.log(-jnp.log(u))
    return g + x


def _group_kernel(x_ref, o_ref, *, rows_per_grp, sub, chunk, vocab, lanes):
    grp = pl.program_id(0)
    t = rows_per_grp * sub
    nfull = chunk // lanes
    tail = chunk - nfull * lanes

    qi = jax.lax.broadcasted_iota(jnp.uint32, (t, lanes), 0)
    li = jax.lax.broadcasted_iota(jnp.uint32, (t, lanes), 1)
    q = qi // np.uint32(sub)
    s = qi % np.uint32(sub)
    pat = q * np.uint32(vocab) + s * np.uint32(chunk) + li + _KS1
    vpat = (s * np.uint32(chunk) + li).astype(jnp.int32)

    base = jnp.uint32(grp) * np.uint32(rows_per_grp * vocab)
    big = np.int32(vocab)

    run_max = jnp.full((t, lanes), _NEG_INF, jnp.float32)
    run_idx = jnp.full((t, lanes), big, jnp.int32)
    for c in range(nfull):
        off = c * lanes
        x = x_ref[0, :, off:off + lanes]
        y = _gumbel_plus(x, pat + (base + np.uint32(off)))
        v = vpat + np.int32(off)
        upd = y > run_max
        run_idx = jnp.where(upd, v, run_idx)
        run_max = jnp.maximum(run_max, y)

    off_t = nfull * lanes
    x_t = x_ref[0, :, pl.ds(off_t, tail)]
    y_t = _gumbel_plus(x_t, pat[:, :tail] + (base + np.uint32(off_t)))
    v_t = vpat[:, :tail] + np.int32(off_t)
    pad = lanes - tail
    y_full = jnp.concatenate(
        [y_t, jnp.full((t, pad), _NEG_INF, jnp.float32)], axis=1)
    v_full = jnp.concatenate([v_t, jnp.full((t, pad), big, jnp.int32)], axis=1)
    upd_t = y_full > run_max
    run_idx = jnp.where(upd_t, v_full, run_idx)
    run_max = jnp.maximum(run_max, y_full)

    acts = []
    for r in range(rows_per_grp):
        rm = run_max[r * sub:(r + 1) * sub]
        ri = run_idx[r * sub:(r + 1) * sub]
        m = jnp.max(rm)
        acts.append(jnp.min(jnp.where(rm == m, ri, big)))

    a32 = jnp.full((t, 1), acts[0], jnp.int32)
    qcol = jax.lax.broadcasted_iota(jnp.int32, (t, 1), 0) // np.int32(sub)
    for r in range(1, rows_per_grp):
        a32 = jnp.where(qcol == r, acts[r], a32)

    one = np.float32(1.0)
    zero = np.float32(0.0)

    for c in range(nfull):
        off = c * lanes
        v = vpat + np.int32(off)
        o_ref[0, :, off:off + lanes] = jnp.where(v == a32, one, zero)
    v_tl = vpat[:, :tail] + np.int32(off_t)
    o_ref[0, :, pl.ds(off_t, tail)] = jnp.where(v_tl == a32, one, zero)


def kernel(x):
    m, n, vocab = x.shape
    rows = m * n
    sub = 8
    rows_per_grp = 4
    grps = rows // rows_per_grp
    chunk = vocab // sub
    t = rows_per_grp * sub
    xr = x.reshape(grps, t, chunk)
    out = pl.pallas_call(
        functools.partial(_group_kernel, rows_per_grp=rows_per_grp, sub=sub,
                          chunk=chunk, vocab=vocab, lanes=512),
        grid=(grps,),
        in_specs=[pl.BlockSpec((1, t, chunk), lambda g: (g, 0, 0))],
        out_specs=pl.BlockSpec((1, t, chunk), lambda g: (g, 0, 0)),
        out_shape=jax.ShapeDtypeStruct((grps, t, chunk), jnp.float32),
        compiler_params=pltpu.CompilerParams(
            dimension_semantics=("parallel",)),
    )(xr)
    return out.reshape(m, n, vocab)
